# baseline (speedup 1.0000x reference)
import functools

import jax
import jax.numpy as jnp
from jax import lax
from jax.experimental import pallas as pl
from jax.experimental.pallas import tpu as pltpu
from jax.experimental.pallas import tpu_sc as plsc

VOCAB = 1_000_000
D = 64
B = 4096
S = 200
H = 100
NC, NS = 2, 16
NW = NC * NS
EPW = B // NW
RPW = 2 * EPW
NLAB = 10


def _sc_pool(table, x2):
    mesh = plsc.VectorSubcoreMesh(core_axis_name="c", subcore_axis_name="s")

    @functools.partial(
        pl.kernel,
        out_type=jax.ShapeDtypeStruct((B, D), jnp.float32),
        mesh=mesh,
        compiler_params=pltpu.CompilerParams(use_tc_tiling_on_sc=False),
        scratch_types=[
            pltpu.VMEM((RPW, H), jnp.int32),
            pltpu.VMEM((4, H, D), jnp.float32),
            pltpu.VMEM((EPW, D), jnp.float32),
            pltpu.SemaphoreType.DMA((4,)),
        ],
    )
    def k(table_hbm, x_hbm, out_hbm, idx_v, bufs, bow_v, sems):
        wid = lax.axis_index("s") * NC + lax.axis_index("c")
        row0 = wid * RPW
        pltpu.sync_copy(x_hbm.at[pl.ds(row0, RPW)], idx_v)

        def fire(e, h, slot):
            pltpu.async_copy(
                table_hbm.at[idx_v.at[2 * e + h]], bufs.at[slot], sems.at[slot]
            )

        def wait(e, h, slot):
            pltpu.make_async_copy(
                table_hbm.at[idx_v.at[2 * e + h]], bufs.at[slot], sems.at[slot]
            ).wait()

        for p in range(2):
            for h in range(2):
                fire(p, h, 2 * p + h)

        scale = jnp.float32(1.0 / S)

        def eloop(i, _):
            for p in range(2):
                e = i * 2 + p
                acc = (jnp.zeros((16,), jnp.float32),) * 4
                for h in range(2):
                    slot = 2 * p + h
                    wait(e, h, slot)

                    def tbody(t, a, _slot=slot):
                        return tuple(
                            a[j] + bufs[_slot, t, pl.ds(16 * j, 16)]
                            for j in range(4)
                        )

                    acc = lax.fori_loop(0, H, tbody, acc)
                    ne = jnp.minimum(e + 2, EPW - 1)
                    fire(ne, h, slot)
                for j in range(4):
                    bow_v[e, pl.ds(16 * j, 16)] = acc[j] * scale
            return 0

        lax.fori_loop(0, EPW // 2, eloop, 0)

        for p in range(2):
            for h in range(2):
                wait(EPW - 1, h, 2 * p + h)

        pltpu.sync_copy(bow_v, out_hbm.at[pl.ds(wid * EPW, EPW)])

    return k(table, x2)


def _tc_head(bow, W, b):
    def body(bow_ref, w_ref, b_ref, out_ref):
        logits = (
            jnp.dot(bow_ref[...], w_ref[...], preferred_element_type=jnp.float32)
            + b_ref[...]
        )
        m = jnp.max(logits, axis=1, keepdims=True)
        s = logits - m
        lse = jnp.log(jnp.sum(jnp.exp(s), axis=1, keepdims=True))
        out_ref[...] = s - lse

    return pl.pallas_call(
        body,
        out_shape=jax.ShapeDtypeStruct((B, NLAB), jnp.float32),
    )(bow, W, b.reshape(1, NLAB))


@jax.jit
def kernel(x, table, W, b):
    x2 = x.reshape(2 * B, H).astype(jnp.int32)
    bow = _sc_pool(table, x2)
    return _tc_head(bow, W, b)


# speedup vs baseline: 1.0097x; 1.0097x over previous
import functools

import jax
import jax.numpy as jnp
from jax import lax
from jax.experimental import pallas as pl
from jax.experimental.pallas import tpu as pltpu
from jax.experimental.pallas import tpu_sc as plsc

VOCAB = 1_000_000
D = 64
B = 4096
S = 200
H = 100
NC, NS = 2, 16
NW = NC * NS
EPW = B // NW
RPW = 2 * EPW
NLAB = 10


def _sc_pool(table, x2):
    mesh = plsc.VectorSubcoreMesh(core_axis_name="c", subcore_axis_name="s")

    @functools.partial(
        pl.kernel,
        out_type=jax.ShapeDtypeStruct((B, D), jnp.float32),
        mesh=mesh,
        compiler_params=pltpu.CompilerParams(use_tc_tiling_on_sc=False),
        scratch_types=[
            pltpu.VMEM((RPW, H), jnp.int32),
            pltpu.VMEM((4, H, D), jnp.float32),
            pltpu.VMEM((EPW, D), jnp.float32),
            pltpu.SemaphoreType.DMA((4,)),
        ],
    )
    def k(table_hbm, x_hbm, out_hbm, idx_v, bufs, bow_v, sems):
        wid = lax.axis_index("s") * NC + lax.axis_index("c")
        row0 = wid * RPW
        pltpu.sync_copy(x_hbm.at[pl.ds(row0, RPW)], idx_v)

        def fire(e, h, slot):
            pltpu.async_copy(
                table_hbm.at[idx_v.at[2 * e + h]], bufs.at[slot], sems.at[slot]
            )

        def wait(e, h, slot):
            pltpu.make_async_copy(
                table_hbm.at[idx_v.at[2 * e + h]], bufs.at[slot], sems.at[slot]
            ).wait()

        for p in range(2):
            for h in range(2):
                fire(p, h, 2 * p + h)

        scale = jnp.float32(1.0 / S)

        def eloop(i, _):
            for p in range(2):
                e = i * 2 + p
                acc = (jnp.zeros((16,), jnp.float32),) * 8
                for h in range(2):
                    slot = 2 * p + h
                    wait(e, h, slot)

                    def tbody(t, a, _slot=slot):
                        new = list(a)
                        base = t * 4
                        for u in range(4):
                            for j in range(4):
                                new[(u % 2) * 4 + j] = (
                                    new[(u % 2) * 4 + j]
                                    + bufs[_slot, base + u, pl.ds(16 * j, 16)]
                                )
                        return tuple(new)

                    acc = lax.fori_loop(0, H // 4, tbody, acc)
                    ne = jnp.minimum(e + 2, EPW - 1)
                    fire(ne, h, slot)
                for j in range(4):
                    bow_v[e, pl.ds(16 * j, 16)] = (acc[j] + acc[4 + j]) * scale
            return 0

        lax.fori_loop(0, EPW // 2, eloop, 0)

        for p in range(2):
            for h in range(2):
                wait(EPW - 1, h, 2 * p + h)

        pltpu.sync_copy(bow_v, out_hbm.at[pl.ds(wid * EPW, EPW)])

    return k(table, x2)


def _tc_head(bow, W, b):
    def body(bow_ref, w_ref, b_ref, out_ref):
        logits = (
            jnp.dot(bow_ref[...], w_ref[...], preferred_element_type=jnp.float32)
            + b_ref[...]
        )
        m = jnp.max(logits, axis=1, keepdims=True)
        s = logits - m
        lse = jnp.log(jnp.sum(jnp.exp(s), axis=1, keepdims=True))
        out_ref[...] = s - lse

    return pl.pallas_call(
        body,
        out_shape=jax.ShapeDtypeStruct((B, NLAB), jnp.float32),
    )(bow, W, b.reshape(1, NLAB))


@jax.jit
def kernel(x, table, W, b):
    x2 = x.reshape(2 * B, H).astype(jnp.int32)
    bow = _sc_pool(table, x2)
    return _tc_head(bow, W, b)
